# Initial kernel scaffold; baseline (speedup 1.0000x reference)
import jax
import jax.numpy as jnp
from jax.experimental import pallas as pl


def kernel(x, embed_in, embed_in_2, lin0_w, lin0_b):
    raise NotImplementedError("write your pallas kernel here")


# speedup vs baseline: 168.2231x; 168.2231x over previous
import functools

import jax
import jax.numpy as jnp
from jax import lax
from jax.experimental import pallas as pl
from jax.experimental.pallas import tpu as pltpu
from jax.experimental.pallas import tpu_sc as plsc

NC, NS, L = 2, 16, 16
NW = NC * NS
BATCH, SEQ, DIM = 16384, 200, 10
N = BATCH * SEQ
PER_TILE = N // NW
CHUNK = 25_600
NCHUNK = PER_TILE // CHUNK
TBL = 128

_mesh = plsc.VectorSubcoreMesh(core_axis_name="c", subcore_axis_name="s")


@functools.partial(
    pl.kernel,
    out_type=jax.ShapeDtypeStruct((N,), jnp.float32),
    mesh=_mesh,
    compiler_params=pltpu.CompilerParams(needs_layout_passes=False),
    scratch_types=[
        pltpu.VMEM((DIM, TBL), jnp.float32),
        pltpu.VMEM((DIM, TBL), jnp.float32),
        pltpu.VMEM((L,), jnp.float32),
        pltpu.VMEM((TBL,), jnp.float32),
        pltpu.VMEM((CHUNK,), jnp.int32),
        pltpu.VMEM((CHUNK,), jnp.float32),
    ],
)
def _embed_gather(e1t_hbm, e2t_hbm, wb_hbm, idx_hbm, out_hbm,
                  e1t_v, e2t_v, wb_v, table_v, idx_v, out_v):
    pltpu.sync_copy(e1t_hbm, e1t_v)
    pltpu.sync_copy(e2t_hbm, e2t_v)
    pltpu.sync_copy(wb_hbm, wb_v)
    wbv = wb_v[...]
    for c in range(TBL // L):
        acc = jnp.zeros((L,), jnp.float32) + wbv[DIM]
        for d in range(DIM):
            acc = acc + (e1t_v[d, pl.ds(c * L, L)]
                         + e2t_v[d, pl.ds(c * L, L)]) * wbv[d]
        table_v[pl.ds(c * L, L)] = acc

    wid = lax.axis_index("s") * NC + lax.axis_index("c")
    base = wid * PER_TILE
    for ci in range(NCHUNK):
        off = base + ci * CHUNK
        pltpu.sync_copy(idx_hbm.at[pl.ds(off, CHUNK)], idx_v)

        def body(i, carry):
            j = pl.multiple_of(i * L, L)
            iv = idx_v[pl.ds(j, L)]
            out_v[pl.ds(j, L)] = plsc.load_gather(table_v, [iv])
            return carry

        lax.fori_loop(0, CHUNK // L, body, 0)
        pltpu.sync_copy(out_v, out_hbm.at[pl.ds(off, CHUNK)])


def kernel(x, embed_in, embed_in_2, lin0_w, lin0_b):
    idx = x.reshape(-1).astype(jnp.int32)
    e1t = jnp.zeros((DIM, TBL), jnp.float32).at[:, :100].set(embed_in.T)
    e2t = jnp.zeros((DIM, TBL), jnp.float32).at[:, :100].set(embed_in_2.T)
    wb = jnp.concatenate(
        [lin0_w[0], lin0_b, jnp.zeros((L - DIM - 1,), jnp.float32)])
    out = _embed_gather(e1t, e2t, wb, idx)
    return out.reshape(BATCH, SEQ, 1)


# speedup vs baseline: 215.6244x; 1.2818x over previous
import functools

import jax
import jax.numpy as jnp
from jax import lax
from jax.experimental import pallas as pl
from jax.experimental.pallas import tpu as pltpu
from jax.experimental.pallas import tpu_sc as plsc

NC, NS, L = 2, 16, 16
NW = NC * NS
BATCH, SEQ, DIM = 16384, 200, 10
N = BATCH * SEQ
PER_TILE = N // NW
CHUNK = 12_800
NCHUNK = PER_TILE // CHUNK
TBL = 128

_mesh = plsc.VectorSubcoreMesh(core_axis_name="c", subcore_axis_name="s")


@functools.partial(
    pl.kernel,
    out_type=jax.ShapeDtypeStruct((N,), jnp.float32),
    mesh=_mesh,
    compiler_params=pltpu.CompilerParams(needs_layout_passes=False),
    scratch_types=[
        pltpu.VMEM((DIM, TBL), jnp.float32),
        pltpu.VMEM((DIM, TBL), jnp.float32),
        pltpu.VMEM((L,), jnp.float32),
        pltpu.VMEM((TBL,), jnp.float32),
        pltpu.VMEM((2, CHUNK), jnp.int32),
        pltpu.VMEM((2, CHUNK), jnp.float32),
        pltpu.SemaphoreType.DMA,
        pltpu.SemaphoreType.DMA,
        pltpu.SemaphoreType.DMA,
        pltpu.SemaphoreType.DMA,
    ],
)
def _embed_gather(e1t_hbm, e2t_hbm, wb_hbm, idx_hbm, out_hbm,
                  e1t_v, e2t_v, wb_v, table_v, idx_v, out_v,
                  in_sem0, in_sem1, out_sem0, out_sem1):
    in_sems = (in_sem0, in_sem1)
    out_sems = (out_sem0, out_sem1)
    wid = lax.axis_index("s") * NC + lax.axis_index("c")
    base = wid * PER_TILE

    def in_copy(ci):
        b = ci % 2
        off = base + ci * CHUNK
        return pltpu.make_async_copy(
            idx_hbm.at[pl.ds(off, CHUNK)], idx_v.at[b], in_sems[b])

    def out_copy(ci):
        b = ci % 2
        off = base + ci * CHUNK
        return pltpu.make_async_copy(
            out_v.at[b], out_hbm.at[pl.ds(off, CHUNK)], out_sems[b])

    in_copy(0).start()
    in_copy(1).start()

    pltpu.sync_copy(e1t_hbm, e1t_v)
    pltpu.sync_copy(e2t_hbm, e2t_v)
    pltpu.sync_copy(wb_hbm, wb_v)
    wbv = wb_v[...]
    for c in range(TBL // L):
        acc = jnp.zeros((L,), jnp.float32) + wbv[DIM]
        for d in range(DIM):
            acc = acc + (e1t_v[d, pl.ds(c * L, L)]
                         + e2t_v[d, pl.ds(c * L, L)]) * wbv[d]
        table_v[pl.ds(c * L, L)] = acc

    for ci in range(NCHUNK):
        b = ci % 2
        in_copy(ci).wait()
        if ci >= 2:
            out_copy(ci - 2).wait()

        @plsc.parallel_loop(0, CHUNK, step=L, unroll=8)
        def _(i):
            iv = idx_v[b, pl.ds(i, L)]
            out_v[b, pl.ds(i, L)] = plsc.load_gather(table_v, [iv])

        out_copy(ci).start()
        if ci + 2 < NCHUNK:
            in_copy(ci + 2).start()
    out_copy(NCHUNK - 2).wait()
    out_copy(NCHUNK - 1).wait()


def kernel(x, embed_in, embed_in_2, lin0_w, lin0_b):
    idx = x.reshape(-1).astype(jnp.int32)
    e1t = jnp.zeros((DIM, TBL), jnp.float32).at[:, :100].set(embed_in.T)
    e2t = jnp.zeros((DIM, TBL), jnp.float32).at[:, :100].set(embed_in_2.T)
    wb = jnp.concatenate(
        [lin0_w[0], lin0_b, jnp.zeros((L - DIM - 1,), jnp.float32)])
    out = _embed_gather(e1t, e2t, wb, idx)
    return out.reshape(BATCH, SEQ, 1)


# speedup vs baseline: 426.3207x; 1.9771x over previous
import functools

import jax
import jax.numpy as jnp
from jax import lax
from jax.experimental import pallas as pl
from jax.experimental.pallas import tpu as pltpu
from jax.experimental.pallas import tpu_sc as plsc

NC, NS, L = 2, 16, 16
NW = NC * NS
BATCH, SEQ, DIM = 16384, 200, 10
N = BATCH * SEQ
LB = SEQ // 8
BT = BATCH // 128
BT_W = BT // NW
SEG = BT_W * 8 * 128
TBL = 128

_mesh = plsc.VectorSubcoreMesh(core_axis_name="c", subcore_axis_name="s")


@functools.partial(
    pl.kernel,
    out_type=jax.ShapeDtypeStruct((SEQ, BATCH), jnp.float32),
    mesh=_mesh,
    compiler_params=pltpu.CompilerParams(needs_layout_passes=False),
    scratch_types=[
        pltpu.VMEM((DIM, TBL), jnp.float32),
        pltpu.VMEM((DIM, TBL), jnp.float32),
        pltpu.VMEM((L,), jnp.float32),
        pltpu.VMEM((TBL,), jnp.float32),
        pltpu.VMEM((2, SEG), jnp.int32),
        pltpu.VMEM((2, 8, BT_W * 128), jnp.float32),
        pltpu.SemaphoreType.DMA,
        pltpu.SemaphoreType.DMA,
        pltpu.SemaphoreType.DMA,
        pltpu.SemaphoreType.DMA,
    ],
)
def _embed_gather(e1t_hbm, e2t_hbm, wb_hbm, xp_hbm, out_hbm,
                  e1t_v, e2t_v, wb_v, table_v, ib, ob,
                  in_sem0, in_sem1, out_sem0, out_sem1):
    in_sems = (in_sem0, in_sem1)
    out_sems = (out_sem0, out_sem1)
    wid = lax.axis_index("s") * NC + lax.axis_index("c")
    bt0 = wid * BT_W

    def in_copy(lb, b):
        off = lb * (BT * 1024) + bt0 * 1024
        return pltpu.make_async_copy(
            xp_hbm.at[pl.ds(off, SEG)], ib.at[b], in_sems[b])

    def out_copy(lb, b):
        return pltpu.make_async_copy(
            ob.at[b],
            out_hbm.at[pl.ds(lb * 8, 8), pl.ds(bt0 * 128, BT_W * 128)],
            out_sems[b])

    in_copy(0, 0).start()
    in_copy(1, 1).start()

    pltpu.sync_copy(e1t_hbm, e1t_v)
    pltpu.sync_copy(e2t_hbm, e2t_v)
    pltpu.sync_copy(wb_hbm, wb_v)
    wbv = wb_v[...]
    for c in range(TBL // L):
        acc = jnp.zeros((L,), jnp.float32) + wbv[DIM]
        for d in range(DIM):
            acc = acc + (e1t_v[d, pl.ds(c * L, L)]
                         + e2t_v[d, pl.ds(c * L, L)]) * wbv[d]
        table_v[pl.ds(c * L, L)] = acc

    def pair(g, carry):
        for b in range(2):
            lb = g * 2 + b

            @pl.when(lb < LB)
            def _():
                in_copy(lb, b).wait()

                @pl.when(lb >= 2)
                def _():
                    out_copy(lb - 2, b).wait()

                @plsc.parallel_loop(0, BT_W * 128, step=L, unroll=2)
                def _(j):
                    for r in range(8):
                        src = ((j >> 7) << 10) + r * 128 + (j & 127)
                        iv = ib[b, pl.ds(src, L)]
                        ob[b, r, pl.ds(j, L)] = plsc.load_gather(
                            table_v, [iv])

                out_copy(lb, b).start()

                @pl.when(lb + 2 < LB)
                def _():
                    in_copy(lb + 2, b).start()
        return carry

    lax.fori_loop(0, (LB + 1) // 2, pair, 0)
    out_copy(LB - 2, (LB - 2) % 2).wait()
    out_copy(LB - 1, (LB - 1) % 2).wait()


def kernel(x, embed_in, embed_in_2, lin0_w, lin0_b):
    xp = (x.astype(jnp.int32)
           .reshape(BT, 128, LB, 8)
           .transpose(2, 0, 3, 1)
           .reshape(N))
    e1t = jnp.zeros((DIM, TBL), jnp.float32).at[:, :100].set(embed_in.T)
    e2t = jnp.zeros((DIM, TBL), jnp.float32).at[:, :100].set(embed_in_2.T)
    wb = jnp.concatenate(
        [lin0_w[0], lin0_b, jnp.zeros((L - DIM - 1,), jnp.float32)])
    out = _embed_gather(e1t, e2t, wb, xp)
    return out.transpose(1, 0).reshape(BATCH, SEQ, 1)
